# baseline (speedup 1.0000x reference)
import functools

import jax
import jax.numpy as jnp
from jax import lax
from jax.experimental import pallas as pl
from jax.experimental.pallas import tpu as pltpu
from jax.experimental.pallas import tpu_sc as plsc

BB = 320000
FF = 144
DD = 128

NC = 2
NS = 16
NW = NC * NS

_SC_MESH = dict(core_axis_name="c", subcore_axis_name="s")


def _wid():
    return lax.axis_index("s") * NC + lax.axis_index("c")


GK = 80
G_PER_W = BB // NW
G_NBLK = G_PER_W // GK


def _gather_body(nei_hbm, mess_hbm, out_hbm, idx_v, rows_v, sem):
    base = _wid() * G_PER_W

    def body(j, carry):
        off = base + j * GK
        pltpu.sync_copy(nei_hbm.at[pl.ds(off, GK)], idx_v)
        pltpu.async_copy(mess_hbm.at[idx_v], rows_v, sem).wait()
        pltpu.sync_copy(rows_v, out_hbm.at[pl.ds(off, GK)])
        return carry

    lax.fori_loop(0, G_NBLK, body, 0)


@jax.jit
def _sc_gather(nei_idx, mess):
    return pl.kernel(
        _gather_body,
        out_type=jax.ShapeDtypeStruct((BB, DD), jnp.float32),
        mesh=plsc.VectorSubcoreMesh(**_SC_MESH),
        compiler_params=pltpu.CompilerParams(needs_layout_passes=False),
        scratch_types=[
            pltpu.VMEM((GK,), jnp.int32),
            pltpu.VMEM((GK, DD), jnp.float32),
            pltpu.SemaphoreType.DMA,
        ],
    )(nei_idx, mess)

ROWS_A = 2560
ROWS_B = 2560


CSHIFT = 13
CHUNK = 1 << CSHIFT
NCHUNK = 40
NCP = 48
CAP = 8960
AK = 112
A_NBLK = 5
PB = 80
P_NBLK = G_PER_W // PB


def _hist_body(src_hbm, counts_hbm, srcb, hist, counts_v, sem):
    del sem
    wid = _wid()
    i16 = lax.iota(jnp.int32, 16)
    zeros = jnp.zeros((16,), jnp.int32)
    ones = jnp.ones((16,), jnp.int32)
    for i in range(16 * NCP // 16):
        hist[pl.ds(i * 16, 16)] = zeros

    stripe0 = wid * G_PER_W

    def blk(bi, carry):
        pltpu.sync_copy(src_hbm.at[pl.ds(stripe0 + bi * 2000, 2000)], srcb)

        def vec(v, c2):
            p = srcb[pl.ds(v * 16, 16)]
            c = p >> CSHIFT
            plsc.addupdate_scatter(hist, [i16 * NCP + c], ones)
            return c2

        lax.fori_loop(0, 125, vec, 0)
        return carry

    lax.fori_loop(0, 5, blk, 0)

    for cb in range(NCP // 16):
        acc = jnp.zeros((16,), jnp.int32)
        for l in range(16):
            acc = acc + hist[pl.ds(l * NCP + cb * 16, 16)]
        counts_v[pl.ds(cb * 16, 16)] = acc
    pltpu.sync_copy(counts_v, counts_hbm.at[pl.ds(wid * NCP, NCP)])


@jax.jit
def _sc_hist(src_idx):
    return pl.kernel(
        _hist_body,
        out_type=jax.ShapeDtypeStruct((NW * NCP,), jnp.int32),
        mesh=plsc.VectorSubcoreMesh(**_SC_MESH),
        compiler_params=pltpu.CompilerParams(needs_layout_passes=False),
        scratch_types=[
            pltpu.VMEM((2000,), jnp.int32),
            pltpu.VMEM((16 * NCP,), jnp.int32),
            pltpu.VMEM((NCP,), jnp.int32),
            pltpu.SemaphoreType.DMA,
        ],
    )(src_idx)


def _perm_body(src_hbm, counts_hbm, perm_hbm, countsb, cntref, srcb, posb,
               valb, shbuf, sem):
    wid = _wid()
    i16 = lax.iota(jnp.int32, 16)
    pltpu.sync_copy(counts_hbm, countsb)
    shbuf[pl.ds(56, 16)] = jnp.full((16,), -1, jnp.int32)

    for cb in range(NCP // 16):
        def acc_body(w, a):
            return a + countsb[pl.ds(w * NCP + cb * 16, 16)]

        pw = lax.fori_loop(0, wid, acc_body, jnp.zeros((16,), jnp.int32))
        cntref[pl.ds(cb * 16, 16)] = (i16 + cb * 16) * CAP + pw

    stripe0 = wid * G_PER_W

    def blk(bi, carry):
        base = stripe0 + bi * PB
        pltpu.sync_copy(src_hbm.at[pl.ds(base, PB)], srcb)
        for v in range(PB // 16):
            p = srcb[pl.ds(v * 16, 16)]
            c = p >> CSHIFT
            ck, lane = plsc.sort_key_val(c, i16)
            shbuf[pl.ds(0, 16)] = ck
            shbuf[pl.ds(1, 16)] = ck
            prev = shbuf[pl.ds(0, 16)]
            shbuf[pl.ds(40, 16)] = ck
            nxt = shbuf[pl.ds(41, 16)]
            change = jnp.not_equal(ck, prev)
            start = plsc.cummax(jnp.where(change, i16, jnp.zeros((16,), jnp.int32)))
            rank = i16 - start
            prior = plsc.load_gather(cntref, [ck])
            pos = prior + rank
            plsc.store_scatter(cntref, [ck], pos + 1, mask=jnp.not_equal(ck, nxt))
            posb[pl.ds(v * 16, 16)] = pos
            valb[pl.ds(v * 16, 16)] = base + v * 16 + lane
        pltpu.async_copy(valb, perm_hbm.at[posb], sem).wait()
        return carry

    lax.fori_loop(0, P_NBLK, blk, 0)


@jax.jit
def _sc_perm(src_idx, counts):
    return pl.kernel(
        _perm_body,
        out_type=jax.ShapeDtypeStruct((NCHUNK * CAP,), jnp.int32),
        mesh=plsc.VectorSubcoreMesh(**_SC_MESH),
        compiler_params=pltpu.CompilerParams(needs_layout_passes=False),
        scratch_types=[
            pltpu.VMEM((NW * NCP,), jnp.int32),
            pltpu.VMEM((NCP,), jnp.int32),
            pltpu.VMEM((PB,), jnp.int32),
            pltpu.VMEM((PB,), jnp.int32),
            pltpu.VMEM((PB,), jnp.int32),
            pltpu.VMEM((72,), jnp.int32),
            pltpu.SemaphoreType.DMA,
        ],
    )(src_idx, counts)


ACC_ROWS = 10240
CPS = NCHUNK // NC


def _accum_body(use_nei, *refs):
    if use_nei:
        (perm_hbm, counts_hbm, src_hbm, nei_hbm, val_hbm, zero_hbm, out_hbm,
         countsb, lenbuf, permb, gbuf, srcb, neib, dstb, rowsb, zbuf, acc,
         sem) = refs
    else:
        (perm_hbm, counts_hbm, src_hbm, val_hbm, zero_hbm, out_hbm,
         countsb, lenbuf, permb, gbuf, srcb, dstb, rowsb, zbuf, acc,
         sem) = refs
    ca = lax.axis_index("c")
    t = lax.axis_index("s")
    i16 = lax.iota(jnp.int32, 16)

    pltpu.sync_copy(counts_hbm, countsb)
    pltpu.sync_copy(zero_hbm, zbuf)
    for cb in range(NCP // 16):
        def acc_body(w, a):
            return a + countsb[pl.ds(w * NCP + cb * 16, 16)]

        lenbuf[pl.ds(cb * 16, 16)] = lax.fori_loop(
            0, NW, acc_body, jnp.zeros((16,), jnp.int32))

    def chunk(i, carry):
        c = 2 * i + ca
        lv = lenbuf[pl.ds((c >> 4) * 16, 16)]
        len_c = jnp.sum(jnp.where(i16 == (c & 15), lv, 0))
        lim = c * CAP + len_c
        for z in range(4):
            pltpu.sync_copy(zbuf, acc.at[pl.ds(t * 512 + z * 128, 128)])
        plsc.subcore_barrier()
        for j in range(A_NBLK):
            pos0 = c * CAP + t * (A_NBLK * AK) + j * AK
            pltpu.sync_copy(perm_hbm.at[pl.ds(pos0, AK)], permb)
            for v in range(AK // 16):
                p = permb[pl.ds(v * 16, 16)]
                gbuf[pl.ds(v * 16, 16)] = jnp.minimum(
                    jnp.maximum(p, 0), BB - 1)
            pltpu.async_copy(src_hbm.at[gbuf], srcb, sem).wait()
            if use_nei:
                pltpu.async_copy(nei_hbm.at[gbuf], neib, sem).wait()
                pltpu.async_copy(val_hbm.at[neib], rowsb, sem).wait()
            else:
                pltpu.async_copy(val_hbm.at[gbuf], rowsb, sem).wait()
            for v in range(AK // 16):
                s = srcb[pl.ds(v * 16, 16)]
                relpos = pos0 + v * 16 + i16
                trash = CHUNK + ((t * 16 + i16) & 127)
                dstb[pl.ds(v * 16, 16)] = jnp.where(
                    relpos < lim, s - c * CHUNK, trash)
            pltpu.sync_copy(rowsb, acc.at[dstb], add=True)
        plsc.subcore_barrier()

        @pl.when(c * CHUNK + t * 512 < BB)
        def _flush():
            pltpu.sync_copy(acc.at[pl.ds(t * 512, 512)],
                            out_hbm.at[pl.ds(c * CHUNK + t * 512, 512)])

        plsc.subcore_barrier()
        return carry

    lax.fori_loop(0, CPS, chunk, 0)


def _make_accum(use_nei):
    scratch = [
        pltpu.VMEM((NW * NCP,), jnp.int32),
        pltpu.VMEM((NCP,), jnp.int32),
        pltpu.VMEM((AK,), jnp.int32),
        pltpu.VMEM((AK,), jnp.int32),
        pltpu.VMEM((AK,), jnp.int32),
    ]
    if use_nei:
        scratch.append(pltpu.VMEM((AK,), jnp.int32))
    scratch += [
        pltpu.VMEM((AK,), jnp.int32),
        pltpu.VMEM((AK, DD), jnp.float32),
        pltpu.VMEM((128, DD), jnp.float32),
        pltpu.VMEM_SHARED((ACC_ROWS, DD), jnp.float32),
        pltpu.SemaphoreType.DMA,
    ]

    @jax.jit
    def run(perm, counts, src_idx, *rest):
        return pl.kernel(
            functools.partial(_accum_body, use_nei),
            out_type=jax.ShapeDtypeStruct((BB, DD), jnp.float32),
            mesh=plsc.VectorSubcoreMesh(**_SC_MESH),
            compiler_params=pltpu.CompilerParams(needs_layout_passes=False),
            scratch_types=scratch,
        )(perm, counts, src_idx, *rest,
          jnp.zeros((128, DD), jnp.float32))

    return run


_sc_segsum_nei = _make_accum(True)
_sc_segsum_direct = _make_accum(False)


def _a_body(hk_ref, mk_ref, wr1_ref, wr2_ref, br_ref, rm_ref):
    hk = hk_ref[...]
    mk = mk_ref[...]
    acc = (jnp.dot(hk, wr1_ref[...], preferred_element_type=jnp.float32)
           + jnp.dot(mk, wr2_ref[...], preferred_element_type=jnp.float32)
           + br_ref[...])
    rm_ref[...] = jax.nn.sigmoid(acc) * mk


def _dense_rm(h_ki, mess_ki, Wr_w, Wr_b):
    wr1 = Wr_w[:, :FF].T
    wr2 = Wr_w[:, FF:].T
    br = Wr_b.reshape(1, DD)
    nblk = BB // ROWS_A
    return pl.pallas_call(
        _a_body,
        grid=(nblk,),
        in_specs=[
            pl.BlockSpec((ROWS_A, FF), lambda i: (i, 0)),
            pl.BlockSpec((ROWS_A, DD), lambda i: (i, 0)),
            pl.BlockSpec((FF, DD), lambda i: (0, 0)),
            pl.BlockSpec((DD, DD), lambda i: (0, 0)),
            pl.BlockSpec((1, DD), lambda i: (0, 0)),
        ],
        out_specs=pl.BlockSpec((ROWS_A, DD), lambda i: (i, 0)),
        out_shape=jax.ShapeDtypeStruct((BB, DD), jnp.float32),
    )(h_ki, mess_ki, wr1, wr2, br)


def _b_body(h_ref, s_ref, r_ref, wz1_ref, wz2_ref, bz_ref, ww_ref, bw_ref,
            uw_ref, out_ref):
    h = h_ref[...]
    s = s_ref[...]
    r = r_ref[...]
    z = jax.nn.sigmoid(jnp.dot(h, wz1_ref[...], preferred_element_type=jnp.float32)
                       + jnp.dot(s, wz2_ref[...], preferred_element_type=jnp.float32)
                       + bz_ref[...])
    m = jnp.tanh(jnp.dot(h, ww_ref[...], preferred_element_type=jnp.float32)
                 + bw_ref[...]
                 + jnp.dot(r, uw_ref[...], preferred_element_type=jnp.float32))
    out_ref[...] = (1.0 - z) * s + z * m


def _dense_out(h_ij, s_ij, r_ij, Wz_w, Wz_b, U_w, W_w, W_b):
    wz1 = Wz_w[:, :FF].T
    wz2 = Wz_w[:, FF:].T
    bz = Wz_b.reshape(1, DD)
    ww = W_w.T
    bw = W_b.reshape(1, DD)
    uw = U_w.T
    nblk = BB // ROWS_B
    return pl.pallas_call(
        _b_body,
        grid=(nblk,),
        in_specs=[
            pl.BlockSpec((ROWS_B, FF), lambda i: (i, 0)),
            pl.BlockSpec((ROWS_B, DD), lambda i: (i, 0)),
            pl.BlockSpec((ROWS_B, DD), lambda i: (i, 0)),
            pl.BlockSpec((FF, DD), lambda i: (0, 0)),
            pl.BlockSpec((DD, DD), lambda i: (0, 0)),
            pl.BlockSpec((1, DD), lambda i: (0, 0)),
            pl.BlockSpec((FF, DD), lambda i: (0, 0)),
            pl.BlockSpec((1, DD), lambda i: (0, 0)),
            pl.BlockSpec((DD, DD), lambda i: (0, 0)),
        ],
        out_specs=pl.BlockSpec((ROWS_B, DD), lambda i: (i, 0)),
        out_shape=jax.ShapeDtypeStruct((BB, DD), jnp.float32),
    )(h_ij, s_ij, r_ij, wz1, wz2, bz, ww, bw, uw)


def kernel(h_ij, h_ki, mess, src_idx, nei_idx, Wz_w, Wz_b, Wr_w, Wr_b, U_w,
           W_w, W_b):
    mess_ki = _sc_gather(nei_idx, mess)
    counts = _sc_hist(src_idx)
    perm = _sc_perm(src_idx, counts)

    s_ij = _sc_segsum_nei(perm, counts, src_idx, nei_idx, mess)
    rm = _dense_rm(h_ki, mess_ki, Wr_w, Wr_b)
    r_ij = _sc_segsum_direct(perm, counts, src_idx, rm)
    return _dense_out(h_ij, s_ij, r_ij, Wz_w, Wz_b, U_w, W_w, W_b)


# speedup vs baseline: 1.4176x; 1.4176x over previous
import functools

import jax
import jax.numpy as jnp
from jax import lax
from jax.experimental import pallas as pl
from jax.experimental.pallas import tpu as pltpu
from jax.experimental.pallas import tpu_sc as plsc

BB = 320000
FF = 144
DD = 128

NC = 2
NS = 16
NW = NC * NS

_SC_MESH = dict(core_axis_name="c", subcore_axis_name="s")
_SC_PARAMS = pltpu.CompilerParams(needs_layout_passes=False)


def _wid():
    return lax.axis_index("s") * NC + lax.axis_index("c")


CSHIFT = 13
CHUNK = 1 << CSHIFT
NCHUNK = 40
NCP = 48
CAP = 8960
AK = 112
A_NBLK = 5
PB = 80
G_PER_W = BB // NW
P_NBLK = G_PER_W // PB


def _hist_body(src_hbm, counts_hbm, srcb, hist, counts_v, sem):
    del sem
    wid = _wid()
    i16 = lax.iota(jnp.int32, 16)
    zeros = jnp.zeros((16,), jnp.int32)
    ones = jnp.ones((16,), jnp.int32)
    for i in range(16 * NCP // 16):
        hist[pl.ds(i * 16, 16)] = zeros

    stripe0 = wid * G_PER_W

    def blk(bi, carry):
        pltpu.sync_copy(src_hbm.at[pl.ds(stripe0 + bi * 2000, 2000)], srcb)

        def vec(v, c2):
            p = srcb[pl.ds(v * 16, 16)]
            c = p >> CSHIFT
            plsc.addupdate_scatter(hist, [i16 * NCP + c], ones)
            return c2

        lax.fori_loop(0, 125, vec, 0)
        return carry

    lax.fori_loop(0, 5, blk, 0)

    for cb in range(NCP // 16):
        acc = jnp.zeros((16,), jnp.int32)
        for l in range(16):
            acc = acc + hist[pl.ds(l * NCP + cb * 16, 16)]
        counts_v[pl.ds(cb * 16, 16)] = acc
    pltpu.sync_copy(counts_v, counts_hbm.at[pl.ds(wid * NCP, NCP)])


@jax.jit
def _sc_hist(src_idx):
    return pl.kernel(
        _hist_body,
        out_type=jax.ShapeDtypeStruct((NW * NCP,), jnp.int32),
        mesh=plsc.VectorSubcoreMesh(**_SC_MESH),
        compiler_params=_SC_PARAMS,
        scratch_types=[
            pltpu.VMEM((2000,), jnp.int32),
            pltpu.VMEM((16 * NCP,), jnp.int32),
            pltpu.VMEM((NCP,), jnp.int32),
            pltpu.SemaphoreType.DMA,
        ],
    )(src_idx)


def _perm_body(src_hbm, nei_hbm, counts_hbm, perm_hbm, dst_hbm, neio_hbm,
               countsb, cntref, srcb, neib, posb, valb, dstvb, neivb, shbuf,
               sem_a, sem_b, sem_c):
    wid = _wid()
    i16 = lax.iota(jnp.int32, 16)
    pltpu.sync_copy(counts_hbm, countsb)
    shbuf[pl.ds(56, 16)] = jnp.full((16,), -1, jnp.int32)

    for cb in range(NCP // 16):
        def acc_body(w, a):
            return a + countsb[pl.ds(w * NCP + cb * 16, 16)]

        pw = lax.fori_loop(0, wid, acc_body, jnp.zeros((16,), jnp.int32))
        cntref[pl.ds(cb * 16, 16)] = (i16 + cb * 16) * CAP + pw

    stripe0 = wid * G_PER_W

    def blk(bi, carry):
        base = stripe0 + bi * PB
        pltpu.sync_copy(src_hbm.at[pl.ds(base, PB)], srcb)
        pltpu.sync_copy(nei_hbm.at[pl.ds(base, PB)], neib)
        for v in range(PB // 16):
            p = srcb[pl.ds(v * 16, 16)]
            nv = neib[pl.ds(v * 16, 16)]
            c = p >> CSHIFT
            ck, lane = plsc.sort_key_val(c, i16)
            shbuf[pl.ds(0, 16)] = ck
            shbuf[pl.ds(1, 16)] = ck
            prev = shbuf[pl.ds(0, 16)]
            shbuf[pl.ds(40, 16)] = ck
            nxt = shbuf[pl.ds(41, 16)]
            change = jnp.not_equal(ck, prev)
            start = plsc.cummax(
                jnp.where(change, i16, jnp.zeros((16,), jnp.int32)))
            rank = i16 - start
            prior = plsc.load_gather(cntref, [ck])
            pos = prior + rank
            plsc.store_scatter(cntref, [ck], pos + 1,
                               mask=jnp.not_equal(ck, nxt))
            ps = plsc.load_gather(srcb, [lane + v * 16])
            pn = plsc.load_gather(neib, [lane + v * 16])
            posb[pl.ds(v * 16, 16)] = pos
            valb[pl.ds(v * 16, 16)] = base + v * 16 + lane
            dstvb[pl.ds(v * 16, 16)] = ps & (CHUNK - 1)
            neivb[pl.ds(v * 16, 16)] = pn
        cpa = pltpu.async_copy(valb, perm_hbm.at[posb], sem_a)
        cpb = pltpu.async_copy(dstvb, dst_hbm.at[posb], sem_b)
        cpc = pltpu.async_copy(neivb, neio_hbm.at[posb], sem_c)
        cpa.wait()
        cpb.wait()
        cpc.wait()
        return carry

    lax.fori_loop(0, P_NBLK, blk, 0)


@jax.jit
def _sc_perm(src_idx, nei_idx, counts):
    osh = jax.ShapeDtypeStruct((NCHUNK * CAP,), jnp.int32)
    return pl.kernel(
        _perm_body,
        out_type=(osh, osh, osh),
        mesh=plsc.VectorSubcoreMesh(**_SC_MESH),
        compiler_params=_SC_PARAMS,
        scratch_types=[
            pltpu.VMEM((NW * NCP,), jnp.int32),
            pltpu.VMEM((NCP,), jnp.int32),
            pltpu.VMEM((PB,), jnp.int32),
            pltpu.VMEM((PB,), jnp.int32),
            pltpu.VMEM((PB,), jnp.int32),
            pltpu.VMEM((PB,), jnp.int32),
            pltpu.VMEM((PB,), jnp.int32),
            pltpu.VMEM((PB,), jnp.int32),
            pltpu.VMEM((72,), jnp.int32),
            pltpu.SemaphoreType.DMA,
            pltpu.SemaphoreType.DMA,
            pltpu.SemaphoreType.DMA,
        ],
    )(src_idx, nei_idx, counts)


ACC_ROWS = 10240
CPS = NCHUNK // NC
MKX = BB + 128


def _lens_from_counts(countsb, lenbuf):
    for cb in range(NCP // 16):
        def acc_body(w, a):
            return a + countsb[pl.ds(w * NCP + cb * 16, 16)]

        lenbuf[pl.ds(cb * 16, 16)] = lax.fori_loop(
            0, NW, acc_body, jnp.zeros((16,), jnp.int32))


def _spass_body(perm_hbm, dst_hbm, neio_hbm, counts_hbm, val_hbm, zero_hbm,
                out_hbm, mk_hbm, countsb, lenbuf, permb, dstraw, neibuf, gbuf,
                wbuf, dstb, rowsb, zbuf, acc, sem):
    ca = lax.axis_index("c")
    t = lax.axis_index("s")
    i16 = lax.iota(jnp.int32, 16)

    pltpu.sync_copy(counts_hbm, countsb)
    pltpu.sync_copy(zero_hbm, zbuf)
    _lens_from_counts(countsb, lenbuf)

    def chunk(i, carry):
        c = 2 * i + ca
        lv = lenbuf[pl.ds((c >> 4) * 16, 16)]
        len_c = jnp.sum(jnp.where(i16 == (c & 15), lv, 0))
        lim = c * CAP + len_c
        for z in range(4):
            pltpu.sync_copy(zbuf, acc.at[pl.ds(t * 512 + z * 128, 128)])
        plsc.subcore_barrier()
        for j in range(A_NBLK):
            pos0 = c * CAP + t * (A_NBLK * AK) + j * AK
            cp1 = pltpu.async_copy(perm_hbm.at[pl.ds(pos0, AK)], permb, sem)
            cp2 = pltpu.async_copy(dst_hbm.at[pl.ds(pos0, AK)], dstraw, sem)
            cp3 = pltpu.async_copy(neio_hbm.at[pl.ds(pos0, AK)], neibuf, sem)
            cp1.wait()
            cp2.wait()
            cp3.wait()
            for v in range(AK // 16):
                nv = neibuf[pl.ds(v * 16, 16)]
                gbuf[pl.ds(v * 16, 16)] = jnp.minimum(
                    jnp.maximum(nv, 0), BB - 1)
            pltpu.async_copy(val_hbm.at[gbuf], rowsb, sem).wait()
            for v in range(AK // 16):
                relpos = pos0 + v * 16 + i16
                mask = relpos < lim
                trash = CHUNK + ((t * 16 + i16) & 127)
                dstb[pl.ds(v * 16, 16)] = jnp.where(
                    mask, dstraw[pl.ds(v * 16, 16)], trash)
                pv = permb[pl.ds(v * 16, 16)]
                wtrash = BB + ((t * 16 + i16) & 127)
                wbuf[pl.ds(v * 16, 16)] = jnp.where(
                    mask, jnp.minimum(jnp.maximum(pv, 0), BB - 1), wtrash)
            cpw = pltpu.async_copy(rowsb, mk_hbm.at[wbuf], sem)
            pltpu.sync_copy(rowsb, acc.at[dstb], add=True)
            cpw.wait()
        plsc.subcore_barrier()

        @pl.when(c * CHUNK + t * 512 < BB)
        def _flush():
            pltpu.sync_copy(acc.at[pl.ds(t * 512, 512)],
                            out_hbm.at[pl.ds(c * CHUNK + t * 512, 512)])

        plsc.subcore_barrier()
        return carry

    lax.fori_loop(0, CPS, chunk, 0)


@jax.jit
def _sc_spass(perm, dstloc, neio, counts, mess):
    return pl.kernel(
        _spass_body,
        out_type=(jax.ShapeDtypeStruct((BB, DD), jnp.float32),
                  jax.ShapeDtypeStruct((MKX, DD), jnp.float32)),
        mesh=plsc.VectorSubcoreMesh(**_SC_MESH),
        compiler_params=_SC_PARAMS,
        scratch_types=[
            pltpu.VMEM((NW * NCP,), jnp.int32),
            pltpu.VMEM((NCP,), jnp.int32),
            pltpu.VMEM((AK,), jnp.int32),
            pltpu.VMEM((AK,), jnp.int32),
            pltpu.VMEM((AK,), jnp.int32),
            pltpu.VMEM((AK,), jnp.int32),
            pltpu.VMEM((AK,), jnp.int32),
            pltpu.VMEM((AK,), jnp.int32),
            pltpu.VMEM((AK, DD), jnp.float32),
            pltpu.VMEM((128, DD), jnp.float32),
            pltpu.VMEM_SHARED((ACC_ROWS, DD), jnp.float32),
            pltpu.SemaphoreType.DMA,
        ],
    )(perm, dstloc, neio, counts, mess, jnp.zeros((128, DD), jnp.float32))


def _rpass_body(perm_hbm, dst_hbm, counts_hbm, val_hbm, zero_hbm, out_hbm,
                countsb, lenbuf, permb, dstraw, gbuf, dstb, rowsb, zbuf, acc,
                sem):
    ca = lax.axis_index("c")
    t = lax.axis_index("s")
    i16 = lax.iota(jnp.int32, 16)

    pltpu.sync_copy(counts_hbm, countsb)
    pltpu.sync_copy(zero_hbm, zbuf)
    _lens_from_counts(countsb, lenbuf)

    def chunk(i, carry):
        c = 2 * i + ca
        lv = lenbuf[pl.ds((c >> 4) * 16, 16)]
        len_c = jnp.sum(jnp.where(i16 == (c & 15), lv, 0))
        lim = c * CAP + len_c
        for z in range(4):
            pltpu.sync_copy(zbuf, acc.at[pl.ds(t * 512 + z * 128, 128)])
        plsc.subcore_barrier()
        for j in range(A_NBLK):
            pos0 = c * CAP + t * (A_NBLK * AK) + j * AK
            cp1 = pltpu.async_copy(perm_hbm.at[pl.ds(pos0, AK)], permb, sem)
            cp2 = pltpu.async_copy(dst_hbm.at[pl.ds(pos0, AK)], dstraw, sem)
            cp1.wait()
            cp2.wait()
            for v in range(AK // 16):
                pv = permb[pl.ds(v * 16, 16)]
                gbuf[pl.ds(v * 16, 16)] = jnp.minimum(
                    jnp.maximum(pv, 0), BB - 1)
            pltpu.async_copy(val_hbm.at[gbuf], rowsb, sem).wait()
            for v in range(AK // 16):
                relpos = pos0 + v * 16 + i16
                trash = CHUNK + ((t * 16 + i16) & 127)
                dstb[pl.ds(v * 16, 16)] = jnp.where(
                    relpos < lim, dstraw[pl.ds(v * 16, 16)], trash)
            pltpu.sync_copy(rowsb, acc.at[dstb], add=True)
        plsc.subcore_barrier()

        @pl.when(c * CHUNK + t * 512 < BB)
        def _flush():
            pltpu.sync_copy(acc.at[pl.ds(t * 512, 512)],
                            out_hbm.at[pl.ds(c * CHUNK + t * 512, 512)])

        plsc.subcore_barrier()
        return carry

    lax.fori_loop(0, CPS, chunk, 0)


@jax.jit
def _sc_rpass(perm, dstloc, counts, rm):
    return pl.kernel(
        _rpass_body,
        out_type=jax.ShapeDtypeStruct((BB, DD), jnp.float32),
        mesh=plsc.VectorSubcoreMesh(**_SC_MESH),
        compiler_params=_SC_PARAMS,
        scratch_types=[
            pltpu.VMEM((NW * NCP,), jnp.int32),
            pltpu.VMEM((NCP,), jnp.int32),
            pltpu.VMEM((AK,), jnp.int32),
            pltpu.VMEM((AK,), jnp.int32),
            pltpu.VMEM((AK,), jnp.int32),
            pltpu.VMEM((AK,), jnp.int32),
            pltpu.VMEM((AK, DD), jnp.float32),
            pltpu.VMEM((128, DD), jnp.float32),
            pltpu.VMEM_SHARED((ACC_ROWS, DD), jnp.float32),
            pltpu.SemaphoreType.DMA,
        ],
    )(perm, dstloc, counts, rm, jnp.zeros((128, DD), jnp.float32))


ROWS_A = 2560
ROWS_B = 2560


def _a_body(hk_ref, mk_ref, wr1_ref, wr2_ref, br_ref, rm_ref):
    hk = hk_ref[...]
    mk = mk_ref[...]
    acc = (jnp.dot(hk, wr1_ref[...], preferred_element_type=jnp.float32)
           + jnp.dot(mk, wr2_ref[...], preferred_element_type=jnp.float32)
           + br_ref[...])
    rm_ref[...] = jax.nn.sigmoid(acc) * mk


def _dense_rm(h_ki, mess_ki, Wr_w, Wr_b):
    wr1 = Wr_w[:, :FF].T
    wr2 = Wr_w[:, FF:].T
    br = Wr_b.reshape(1, DD)
    nblk = BB // ROWS_A
    return pl.pallas_call(
        _a_body,
        grid=(nblk,),
        in_specs=[
            pl.BlockSpec((ROWS_A, FF), lambda i: (i, 0)),
            pl.BlockSpec((ROWS_A, DD), lambda i: (i, 0)),
            pl.BlockSpec((FF, DD), lambda i: (0, 0)),
            pl.BlockSpec((DD, DD), lambda i: (0, 0)),
            pl.BlockSpec((1, DD), lambda i: (0, 0)),
        ],
        out_specs=pl.BlockSpec((ROWS_A, DD), lambda i: (i, 0)),
        out_shape=jax.ShapeDtypeStruct((BB, DD), jnp.float32),
    )(h_ki, mess_ki, wr1, wr2, br)


def _b_body(h_ref, s_ref, r_ref, wz1_ref, wz2_ref, bz_ref, ww_ref, bw_ref,
            uw_ref, out_ref):
    h = h_ref[...]
    s = s_ref[...]
    r = r_ref[...]
    z = jax.nn.sigmoid(
        jnp.dot(h, wz1_ref[...], preferred_element_type=jnp.float32)
        + jnp.dot(s, wz2_ref[...], preferred_element_type=jnp.float32)
        + bz_ref[...])
    m = jnp.tanh(jnp.dot(h, ww_ref[...], preferred_element_type=jnp.float32)
                 + bw_ref[...]
                 + jnp.dot(r, uw_ref[...], preferred_element_type=jnp.float32))
    out_ref[...] = (1.0 - z) * s + z * m


def _dense_out(h_ij, s_ij, r_ij, Wz_w, Wz_b, U_w, W_w, W_b):
    wz1 = Wz_w[:, :FF].T
    wz2 = Wz_w[:, FF:].T
    bz = Wz_b.reshape(1, DD)
    ww = W_w.T
    bw = W_b.reshape(1, DD)
    uw = U_w.T
    nblk = BB // ROWS_B
    return pl.pallas_call(
        _b_body,
        grid=(nblk,),
        in_specs=[
            pl.BlockSpec((ROWS_B, FF), lambda i: (i, 0)),
            pl.BlockSpec((ROWS_B, DD), lambda i: (i, 0)),
            pl.BlockSpec((ROWS_B, DD), lambda i: (i, 0)),
            pl.BlockSpec((FF, DD), lambda i: (0, 0)),
            pl.BlockSpec((DD, DD), lambda i: (0, 0)),
            pl.BlockSpec((1, DD), lambda i: (0, 0)),
            pl.BlockSpec((FF, DD), lambda i: (0, 0)),
            pl.BlockSpec((1, DD), lambda i: (0, 0)),
            pl.BlockSpec((DD, DD), lambda i: (0, 0)),
        ],
        out_specs=pl.BlockSpec((ROWS_B, DD), lambda i: (i, 0)),
        out_shape=jax.ShapeDtypeStruct((BB, DD), jnp.float32),
    )(h_ij, s_ij, r_ij, wz1, wz2, bz, ww, bw, uw)


def kernel(h_ij, h_ki, mess, src_idx, nei_idx, Wz_w, Wz_b, Wr_w, Wr_b, U_w,
           W_w, W_b):
    counts = _sc_hist(src_idx)
    perm, dstloc, neio = _sc_perm(src_idx, nei_idx, counts)
    s_ij, mess_ki = _sc_spass(perm, dstloc, neio, counts, mess)
    rm = _dense_rm(h_ki, mess_ki, Wr_w, Wr_b)
    r_ij = _sc_rpass(perm, dstloc, counts, rm)
    return _dense_out(h_ij, s_ij, r_ij, Wz_w, Wz_b, U_w, W_w, W_b)
